# baseline (speedup 1.0000x reference)
import functools

import jax
import jax.numpy as jnp
from jax import lax
from jax.experimental import pallas as pl
from jax.experimental.pallas import tpu as pltpu
from jax.experimental.pallas import tpu_sc as plsc

EMBED_DIM = 64
PAIR = 128
NUM_CORES = 2
NUM_SUBCORES = 16
NUM_WORKERS = NUM_CORES * NUM_SUBCORES
IDX_CHUNK = 128
LANES = 16


def _newton_sqrt(x):
    i = plsc.bitcast(x, jnp.int32)
    y = plsc.bitcast(jnp.int32(0x5F3759DF) - (i >> 1), jnp.float32)
    y = y * (1.5 - 0.5 * x * y * y)
    y = y * (1.5 - 0.5 * x * y * y)
    y = y * (1.5 - 0.5 * x * y * y)
    return jnp.where(x > 0.0, x * y, 0.0)


def _make_sc_kernel(batch):
    bpw = batch // NUM_WORKERS
    n_chunks = bpw // IDX_CHUNK
    mesh = plsc.VectorSubcoreMesh(core_axis_name="c", subcore_axis_name="s")

    @functools.partial(
        pl.kernel,
        mesh=mesh,
        compiler_params=pltpu.CompilerParams(needs_layout_passes=False),
        out_type=jax.ShapeDtypeStruct((batch,), jnp.float32),
        scratch_types=[
            pltpu.VMEM((n_chunks, IDX_CHUNK), jnp.int32),
            pltpu.VMEM((n_chunks, IDX_CHUNK), jnp.int32),
            pltpu.VMEM((n_chunks, IDX_CHUNK), jnp.int32),
            pltpu.VMEM((n_chunks, IDX_CHUNK), jnp.int32),
            pltpu.VMEM((n_chunks, IDX_CHUNK), jnp.int32),
            pltpu.VMEM((n_chunks, IDX_CHUNK), jnp.int32),
            pltpu.VMEM((bpw,), jnp.int32),
            pltpu.VMEM((bpw,), jnp.int32),
            pltpu.VMEM((bpw,), jnp.int32),
            pltpu.VMEM((IDX_CHUNK, PAIR), jnp.float32),
            pltpu.VMEM((IDX_CHUNK, PAIR), jnp.float32),
            pltpu.VMEM((IDX_CHUNK, PAIR), jnp.float32),
            pltpu.VMEM((IDX_CHUNK, PAIR), jnp.float32),
            pltpu.VMEM((IDX_CHUNK, PAIR), jnp.float32),
            pltpu.VMEM((IDX_CHUNK, PAIR), jnp.float32),
            pltpu.VMEM((bpw,), jnp.float32),
            pltpu.SemaphoreType.DMA,
            pltpu.SemaphoreType.DMA,
        ],
    )
    def sc_kernel(ent_hbm, rel_hbm, h_hbm, r_hbm, t_hbm, out_hbm,
                  hidx, ridx, tidx, hg, rg, tg, hoff, roff, toff,
                  hb0, hb1, rb0, rb1, tb0, tb1, outv, sem0, sem1):
        wid = lax.axis_index("s") * NUM_CORES + lax.axis_index("c")
        pltpu.sync_copy(h_hbm.at[pl.ds(wid * n_chunks, n_chunks)], hidx)
        pltpu.sync_copy(r_hbm.at[pl.ds(wid * n_chunks, n_chunks)], ridx)
        pltpu.sync_copy(t_hbm.at[pl.ds(wid * n_chunks, n_chunks)], tidx)

        for raw, gi, off in ((hidx, hg, hoff), (ridx, rg, roff), (tidx, tg, toff)):
            for c in range(n_chunks):
                for k in range(IDX_CHUNK // LANES):
                    v = raw[c, pl.ds(k * LANES, LANES)]
                    gi[c, pl.ds(k * LANES, LANES)] = v >> 1
                    off[pl.ds(c * IDX_CHUNK + k * LANES, LANES)] = (v & 1) << 6

        hbufs, rbufs, tbufs = (hb0, hb1), (rb0, rb1), (tb0, tb1)
        sems = (sem0, sem1)

        def fire(p):
            s = p % 2
            return (
                pltpu.async_copy(ent_hbm.at[hg.at[p]], hbufs[s], sems[s]),
                pltpu.async_copy(rel_hbm.at[rg.at[p]], rbufs[s], sems[s]),
                pltpu.async_copy(ent_hbm.at[tg.at[p]], tbufs[s], sems[s]),
            )

        lane = lax.iota(jnp.int32, LANES)
        inflight = {0: fire(0)}
        for p in range(n_chunks):
            if p + 1 < n_chunks:
                inflight[p + 1] = fire(p + 1)
            for cpy in inflight.pop(p):
                cpy.wait()
            s = p % 2
            hbuf, rbuf, tbuf = hbufs[s], rbufs[s], tbufs[s]

            def group_body(g, carry, hbuf=hbuf, rbuf=rbuf, tbuf=tbuf, p=p):
                row0 = g * LANES
                gbase = p * IDX_CHUNK + row0
                hoffv = hoff[pl.ds(gbase, LANES)]
                roffv = roff[pl.ds(gbase, LANES)]
                toffv = toff[pl.ds(gbase, LANES)]
                vec = jnp.zeros((LANES,), jnp.float32)
                for u in range(LANES):
                    b = row0 + u
                    ho, ro, to = hoffv[u], roffv[u], toffv[u]
                    acc = jnp.zeros((LANES,), jnp.float32)
                    for c2 in range(EMBED_DIM // LANES):
                        d = c2 * LANES
                        diff = (hbuf[b, pl.ds(ho + d, LANES)]
                                + rbuf[b, pl.ds(ro + d, LANES)]
                                - tbuf[b, pl.ds(to + d, LANES)])
                        acc = acc + diff * diff
                    vec = jnp.where(lane == u, jnp.sum(acc), vec)
                outv[pl.ds(gbase, LANES)] = -_newton_sqrt(vec)
                return carry

            lax.fori_loop(0, IDX_CHUNK // LANES, group_body, 0)

        pltpu.sync_copy(outv, out_hbm.at[pl.ds(wid * bpw, bpw)])

    return sc_kernel


def kernel(entity_table, relation_table, h, r, t):
    batch = h.shape[0]
    n_ent, dim = entity_table.shape
    n_rel = relation_table.shape[0]
    ent2 = entity_table.reshape(n_ent * dim // PAIR, PAIR)
    rel2 = relation_table.reshape(n_rel * dim // PAIR, PAIR)
    shape2 = (NUM_WORKERS * (batch // NUM_WORKERS // IDX_CHUNK), IDX_CHUNK)
    h2 = h.astype(jnp.int32).reshape(shape2)
    r2 = r.astype(jnp.int32).reshape(shape2)
    t2 = t.astype(jnp.int32).reshape(shape2)
    return _make_sc_kernel(batch)(ent2, rel2, h2, r2, t2)


# speedup vs baseline: 1.6908x; 1.6908x over previous
import functools

import jax
import jax.numpy as jnp
from jax import lax
from jax.experimental import pallas as pl
from jax.experimental.pallas import tpu as pltpu
from jax.experimental.pallas import tpu_sc as plsc

EMBED_DIM = 64
NUM_CORES = 2
NUM_SUBCORES = 16
NUM_WORKERS = NUM_CORES * NUM_SUBCORES
PASS_ROWS = 128
LANES = 16


def _newton_sqrt(x):
    i = plsc.bitcast(x, jnp.int32)
    y = plsc.bitcast(jnp.int32(0x5F3759DF) - (i >> 1), jnp.float32)
    y = y * (1.5 - 0.5 * x * y * y)
    y = y * (1.5 - 0.5 * x * y * y)
    y = y * (1.5 - 0.5 * x * y * y)
    return jnp.where(x > 0.0, x * y, 0.0)


def _make_sc_kernel(batch):
    bpw = batch // NUM_WORKERS
    n_passes = bpw // PASS_ROWS
    groups = PASS_ROWS // LANES
    mesh = plsc.VectorSubcoreMesh(core_axis_name="c", subcore_axis_name="s")

    @functools.partial(
        pl.kernel,
        mesh=mesh,
        compiler_params=pltpu.CompilerParams(needs_layout_passes=False),
        out_type=jax.ShapeDtypeStruct((batch,), jnp.float32),
        scratch_types=[
            pltpu.VMEM((n_passes, PASS_ROWS), jnp.int32),
            pltpu.VMEM((n_passes, PASS_ROWS), jnp.int32),
            pltpu.VMEM((n_passes, PASS_ROWS), jnp.int32),
            pltpu.VMEM((PASS_ROWS, EMBED_DIM), jnp.float32),
            pltpu.VMEM((PASS_ROWS, EMBED_DIM), jnp.float32),
            pltpu.VMEM((PASS_ROWS, EMBED_DIM), jnp.float32),
            pltpu.VMEM((PASS_ROWS, EMBED_DIM), jnp.float32),
            pltpu.VMEM((PASS_ROWS, EMBED_DIM), jnp.float32),
            pltpu.VMEM((PASS_ROWS, EMBED_DIM), jnp.float32),
            pltpu.VMEM((bpw,), jnp.float32),
            pltpu.SemaphoreType.DMA,
            pltpu.SemaphoreType.DMA,
        ],
    )
    def sc_kernel(ent_hbm, rel_hbm, h_hbm, r_hbm, t_hbm, dummy_hbm, out_hbm,
                  hidx, ridx, tidx, hb0, hb1, rb0, rb1, tb0, tb1, outv,
                  sem0, sem1):
        wid = lax.axis_index("s") * NUM_CORES + lax.axis_index("c")
        pltpu.sync_copy(h_hbm.at[pl.ds(wid * n_passes, n_passes)], hidx)
        pltpu.sync_copy(r_hbm.at[pl.ds(wid * n_passes, n_passes)], ridx)
        pltpu.sync_copy(t_hbm.at[pl.ds(wid * n_passes, n_passes)], tidx)

        hbufs, rbufs, tbufs = (hb0, hb1), (rb0, rb1), (tb0, tb1)
        sems = (sem0, sem1)
        lane = lax.iota(jnp.int32, LANES)

        def fire(p):
            s = p % 2
            hbuf, rbuf, tbuf, sem = hbufs[s], rbufs[s], tbufs[s], sems[s]

            def body(k, carry):
                hv = hidx[p, pl.ds(k * LANES, LANES)]
                rv = ridx[p, pl.ds(k * LANES, LANES)]
                tv = tidx[p, pl.ds(k * LANES, LANES)]
                for u in range(LANES):
                    lrow = k * LANES + u
                    pltpu.async_copy(ent_hbm.at[hv[u]], hbuf.at[lrow], sem)
                    pltpu.async_copy(rel_hbm.at[rv[u]], rbuf.at[lrow], sem)
                    pltpu.async_copy(ent_hbm.at[tv[u]], tbuf.at[lrow], sem)
                return carry

            lax.fori_loop(0, groups, body, 0)

        def drain(p):
            s = p % 2
            for buf in (hbufs[s], rbufs[s], tbufs[s]):
                pltpu.make_async_copy(dummy_hbm, buf, sems[s]).wait()

        fire(0)
        for p in range(n_passes):
            if p + 1 < n_passes:
                fire(p + 1)
            drain(p)
            s = p % 2
            hbuf, rbuf, tbuf = hbufs[s], rbufs[s], tbufs[s]

            def group_body(g, carry, hbuf=hbuf, rbuf=rbuf, tbuf=tbuf, p=p):
                row0 = g * LANES
                vec = jnp.zeros((LANES,), jnp.float32)
                for u in range(LANES):
                    b = row0 + u
                    acc = jnp.zeros((LANES,), jnp.float32)
                    for c in range(EMBED_DIM // LANES):
                        sl = pl.ds(c * LANES, LANES)
                        diff = hbuf[b, sl] + rbuf[b, sl] - tbuf[b, sl]
                        acc = acc + diff * diff
                    vec = jnp.where(lane == u, jnp.sum(acc), vec)
                outv[pl.ds(p * PASS_ROWS + row0, LANES)] = -_newton_sqrt(vec)
                return carry

            lax.fori_loop(0, groups, group_body, 0)

        pltpu.sync_copy(outv, out_hbm.at[pl.ds(wid * bpw, bpw)])

    return sc_kernel


def kernel(entity_table, relation_table, h, r, t):
    batch = h.shape[0]
    shape2 = (NUM_WORKERS * (batch // NUM_WORKERS // PASS_ROWS), PASS_ROWS)
    h2 = h.astype(jnp.int32).reshape(shape2)
    r2 = r.astype(jnp.int32).reshape(shape2)
    t2 = t.astype(jnp.int32).reshape(shape2)
    dummy = jnp.zeros((PASS_ROWS, EMBED_DIM), jnp.float32)
    return _make_sc_kernel(batch)(entity_table, relation_table, h2, r2, t2, dummy)


# speedup vs baseline: 2.4753x; 1.4640x over previous
import functools

import jax
import jax.numpy as jnp
from jax import lax
from jax.experimental import pallas as pl
from jax.experimental.pallas import tpu as pltpu
from jax.experimental.pallas import tpu_sc as plsc

EMBED_DIM = 64
NUM_CORES = 2
NUM_SUBCORES = 16
NUM_WORKERS = NUM_CORES * NUM_SUBCORES
PASS_ROWS = 128
LANES = 16


def _newton_sqrt(x):
    i = plsc.bitcast(x, jnp.int32)
    y = plsc.bitcast(jnp.int32(0x5F3759DF) - (i >> 1), jnp.float32)
    y = y * (1.5 - 0.5 * x * y * y)
    y = y * (1.5 - 0.5 * x * y * y)
    y = y * (1.5 - 0.5 * x * y * y)
    return jnp.where(x > 0.0, x * y, 0.0)


def _make_sc_kernel(batch):
    bpw = batch // NUM_WORKERS
    n_passes = bpw // PASS_ROWS
    groups = PASS_ROWS // LANES
    mesh = plsc.VectorSubcoreMesh(core_axis_name="c", subcore_axis_name="s")

    @functools.partial(
        pl.kernel,
        mesh=mesh,
        compiler_params=pltpu.CompilerParams(needs_layout_passes=False),
        out_type=jax.ShapeDtypeStruct((batch,), jnp.float32),
        scratch_types=[
            pltpu.VMEM((n_passes, PASS_ROWS), jnp.int32),
            pltpu.VMEM((n_passes, PASS_ROWS), jnp.int32),
            pltpu.VMEM((n_passes, PASS_ROWS), jnp.int32),
            pltpu.VMEM((PASS_ROWS, EMBED_DIM), jnp.float32),
            pltpu.VMEM((PASS_ROWS, EMBED_DIM), jnp.float32),
            pltpu.VMEM((PASS_ROWS, EMBED_DIM), jnp.float32),
            pltpu.VMEM((PASS_ROWS, EMBED_DIM), jnp.float32),
            pltpu.VMEM((PASS_ROWS, EMBED_DIM), jnp.float32),
            pltpu.VMEM((PASS_ROWS, EMBED_DIM), jnp.float32),
            pltpu.VMEM((bpw,), jnp.float32),
            pltpu.SemaphoreType.DMA,
            pltpu.SemaphoreType.DMA,
        ],
    )
    def sc_kernel(ent_hbm, rel_hbm, h_hbm, r_hbm, t_hbm, dummy_hbm, out_hbm,
                  hidx, ridx, tidx, hb0, hb1, rb0, rb1, tb0, tb1, outv,
                  sem0, sem1):
        wid = lax.axis_index("s") * NUM_CORES + lax.axis_index("c")
        pltpu.sync_copy(h_hbm.at[pl.ds(wid * n_passes, n_passes)], hidx)
        pltpu.sync_copy(r_hbm.at[pl.ds(wid * n_passes, n_passes)], ridx)
        pltpu.sync_copy(t_hbm.at[pl.ds(wid * n_passes, n_passes)], tidx)

        hbufs, rbufs, tbufs = (hb0, hb1), (rb0, rb1), (tb0, tb1)
        sems = (sem0, sem1)
        lane = lax.iota(jnp.int32, LANES)

        def fire(p):
            s = p % 2
            hbuf, rbuf, tbuf, sem = hbufs[s], rbufs[s], tbufs[s], sems[s]

            def body(k, carry):
                hv = hidx[p, pl.ds(k * LANES, LANES)]
                rv = ridx[p, pl.ds(k * LANES, LANES)]
                tv = tidx[p, pl.ds(k * LANES, LANES)]
                for u in range(LANES):
                    lrow = k * LANES + u
                    pltpu.async_copy(ent_hbm.at[hv[u] >> 3, hv[u] & 7],
                                     hbuf.at[lrow], sem)
                    pltpu.async_copy(rel_hbm.at[rv[u] >> 3, rv[u] & 7],
                                     rbuf.at[lrow], sem)
                    pltpu.async_copy(ent_hbm.at[tv[u] >> 3, tv[u] & 7],
                                     tbuf.at[lrow], sem)
                return carry

            lax.fori_loop(0, groups, body, 0)

        def drain(p):
            s = p % 2
            for buf in (hbufs[s], rbufs[s], tbufs[s]):
                pltpu.make_async_copy(dummy_hbm, buf, sems[s]).wait()

        fire(0)
        for p in range(n_passes):
            if p + 1 < n_passes:
                fire(p + 1)
            drain(p)
            s = p % 2
            hbuf, rbuf, tbuf = hbufs[s], rbufs[s], tbufs[s]

            def group_body(g, carry, hbuf=hbuf, rbuf=rbuf, tbuf=tbuf, p=p):
                row0 = g * LANES
                vec = jnp.zeros((LANES,), jnp.float32)
                for u in range(LANES):
                    b = row0 + u
                    acc = jnp.zeros((LANES,), jnp.float32)
                    for c in range(EMBED_DIM // LANES):
                        sl = pl.ds(c * LANES, LANES)
                        diff = hbuf[b, sl] + rbuf[b, sl] - tbuf[b, sl]
                        acc = acc + diff * diff
                    vec = jnp.where(lane == u, jnp.sum(acc), vec)
                outv[pl.ds(p * PASS_ROWS + row0, LANES)] = -_newton_sqrt(vec)
                return carry

            lax.fori_loop(0, groups, group_body, 0)

        pltpu.sync_copy(outv, out_hbm.at[pl.ds(wid * bpw, bpw)])

    return sc_kernel


def kernel(entity_table, relation_table, h, r, t):
    batch = h.shape[0]
    ent3 = entity_table.reshape(entity_table.shape[0] // 8, 8, EMBED_DIM)
    rel3 = relation_table.reshape(relation_table.shape[0] // 8, 8, EMBED_DIM)
    shape2 = (NUM_WORKERS * (batch // NUM_WORKERS // PASS_ROWS), PASS_ROWS)
    h2 = h.astype(jnp.int32).reshape(shape2)
    r2 = r.astype(jnp.int32).reshape(shape2)
    t2 = t.astype(jnp.int32).reshape(shape2)
    dummy = jnp.zeros((PASS_ROWS, EMBED_DIM), jnp.float32)
    return _make_sc_kernel(batch)(ent3, rel3, h2, r2, t2, dummy)
